# initial kernel scaffold (unmeasured)
import functools

import jax
import jax.numpy as jnp
from jax import lax
from jax.experimental import pallas as pl
from jax.experimental.pallas import tpu as pltpu

N_DEV = 16
M = 4096
K_SHARD = 256
N = 2048
CHUNK = M // N_DEV

PERM = [0, 1, 5, 9, 13, 14, 10, 6, 2, 3, 7, 11, 15, 12, 8, 4]
POS = [0] * N_DEV
for _p, _d in enumerate(PERM):
    POS[_d] = _p


def kernel(x, w_mat, scale_x, scale_w):
    perm_arr = jnp.array(PERM, dtype=jnp.int32)
    pos_arr = jnp.array(POS, dtype=jnp.int32)

    def body(x_ref, w_ref, sx_ref, sw_ref, out_ref,
             acc_ref, send_ref, recv_ref, send_sems, recv_sems):
        d = lax.axis_index("i")
        p = pos_arr[d]
        nxt = perm_arr[(p + 1) % N_DEV]
        prv = perm_arr[(p + N_DEV - 1) % N_DEV]

        barrier_sem = pltpu.get_barrier_semaphore()
        for nbr in (prv, nxt):
            pl.semaphore_signal(
                barrier_sem, inc=1,
                device_id=(nbr,), device_id_type=pl.DeviceIdType.MESH,
            )
        pl.semaphore_wait(barrier_sem, 2)

        def chunk_rows(t):
            c = perm_arr[(p + t) % N_DEV]
            return c * CHUNK

        def partial(t):
            r = chunk_rows(t)
            xc = x_ref[pl.ds(r, CHUNK), :]
            return jnp.dot(xc, w_ref[...], preferred_element_type=jnp.float32)

        acc_ref[...] = partial(N_DEV - 1)

        for s in range(N_DEV - 1):
            send_ref[s, :, :] = acc_ref[...].astype(jnp.bfloat16)
            rdma = pltpu.make_async_remote_copy(
                src_ref=send_ref.at[s],
                dst_ref=recv_ref.at[s],
                send_sem=send_sems.at[s],
                recv_sem=recv_sems.at[s],
                device_id=(nxt,),
                device_id_type=pl.DeviceIdType.MESH,
            )
            rdma.start()
            rdma.wait()
            acc_ref[...] = (
                recv_ref[s, :, :].astype(jnp.float32) + partial(N_DEV - 2 - s)
            )

        scale = sx_ref[0] * sw_ref[0]
        y = acc_ref[...] * scale
        out_ref[...] = y * jax.nn.sigmoid(y)

    return pl.pallas_call(
        body,
        out_shape=jax.ShapeDtypeStruct((CHUNK, N), jnp.float32),
        in_specs=[
            pl.BlockSpec(memory_space=pltpu.VMEM),
            pl.BlockSpec(memory_space=pltpu.VMEM),
            pl.BlockSpec(memory_space=pltpu.SMEM),
            pl.BlockSpec(memory_space=pltpu.SMEM),
        ],
        out_specs=pl.BlockSpec(memory_space=pltpu.VMEM),
        scratch_shapes=[
            pltpu.VMEM((CHUNK, N), jnp.float32),
            pltpu.VMEM((N_DEV - 1, CHUNK, N), jnp.bfloat16),
            pltpu.VMEM((N_DEV - 1, CHUNK, N), jnp.bfloat16),
            pltpu.SemaphoreType.DMA((N_DEV - 1,)),
            pltpu.SemaphoreType.DMA((N_DEV - 1,)),
        ],
        compiler_params=pltpu.CompilerParams(
            collective_id=0,
            vmem_limit_bytes=64 * 1024 * 1024,
        ),
    )(x, w_mat, scale_x, scale_w)


# baseline (device time: 211683 ns/iter reference)
import functools

import jax
import jax.numpy as jnp
from jax import lax
from jax.experimental import pallas as pl
from jax.experimental.pallas import tpu as pltpu

N_DEV = 16
M = 4096
K_SHARD = 256
N = 2048
CHUNK = M // N_DEV

PERM = [0, 1, 5, 9, 13, 14, 10, 6, 2, 3, 7, 11, 15, 12, 8, 4]
POS = [0] * N_DEV
for _p, _d in enumerate(PERM):
    POS[_d] = _p


def kernel(x, w_mat, scale_x, scale_w):
    perm_in = jnp.array(PERM, dtype=jnp.int32)
    pos_in = jnp.array(POS, dtype=jnp.int32)

    def body(x_ref, w_ref, sx_ref, sw_ref, perm_ref, pos_ref, out_ref,
             acc_ref, send_ref, recv_ref, send_sems, recv_sems):
        perm_arr = perm_ref
        d = lax.axis_index("i")
        p = pos_ref[d]
        nxt = perm_arr[(p + 1) % N_DEV]
        prv = perm_arr[(p + N_DEV - 1) % N_DEV]

        barrier_sem = pltpu.get_barrier_semaphore()
        for nbr in (prv, nxt):
            pl.semaphore_signal(
                barrier_sem, inc=1,
                device_id=(nbr,), device_id_type=pl.DeviceIdType.MESH,
            )
        pl.semaphore_wait(barrier_sem, 2)

        def chunk_rows(t):
            c = perm_arr[(p + t) % N_DEV]
            return c * CHUNK

        def partial(t):
            r = chunk_rows(t)
            xc = x_ref[pl.ds(r, CHUNK), :]
            return jnp.dot(xc, w_ref[...], preferred_element_type=jnp.float32)

        acc_ref[...] = partial(N_DEV - 1)

        for s in range(N_DEV - 1):
            send_ref[s, :, :] = acc_ref[...].astype(jnp.bfloat16)
            rdma = pltpu.make_async_remote_copy(
                src_ref=send_ref.at[s],
                dst_ref=recv_ref.at[s],
                send_sem=send_sems.at[s],
                recv_sem=recv_sems.at[s],
                device_id=(nxt,),
                device_id_type=pl.DeviceIdType.MESH,
            )
            rdma.start()
            rdma.wait()
            acc_ref[...] = (
                recv_ref[s, :, :].astype(jnp.float32) + partial(N_DEV - 2 - s)
            )

        scale = sx_ref[0] * sw_ref[0]
        y = acc_ref[...] * scale
        out_ref[...] = y * jax.nn.sigmoid(y)

    return pl.pallas_call(
        body,
        out_shape=jax.ShapeDtypeStruct((CHUNK, N), jnp.float32),
        in_specs=[
            pl.BlockSpec(memory_space=pltpu.VMEM),
            pl.BlockSpec(memory_space=pltpu.VMEM),
            pl.BlockSpec(memory_space=pltpu.SMEM),
            pl.BlockSpec(memory_space=pltpu.SMEM),
            pl.BlockSpec(memory_space=pltpu.SMEM),
            pl.BlockSpec(memory_space=pltpu.SMEM),
        ],
        out_specs=pl.BlockSpec(memory_space=pltpu.VMEM),
        scratch_shapes=[
            pltpu.VMEM((CHUNK, N), jnp.float32),
            pltpu.VMEM((N_DEV - 1, CHUNK, N), jnp.bfloat16),
            pltpu.VMEM((N_DEV - 1, CHUNK, N), jnp.bfloat16),
            pltpu.SemaphoreType.DMA((N_DEV - 1,)),
            pltpu.SemaphoreType.DMA((N_DEV - 1,)),
        ],
        compiler_params=pltpu.CompilerParams(
            collective_id=0,
            vmem_limit_bytes=64 * 1024 * 1024,
        ),
    )(x, w_mat, scale_x, scale_w, perm_in, pos_in)


# device time: 98005 ns/iter; 2.1599x vs baseline; 2.1599x over previous
import jax
import jax.numpy as jnp
from jax import lax
from jax.experimental import pallas as pl
from jax.experimental.pallas import tpu as pltpu

N_DEV = 16
M = 4096
K_SHARD = 256
N = 2048
CHUNK = M // N_DEV
NH = N // 2
B = 2
SUB = NH // B
N_HOP = N_DEV - 1

PERM = [0, 1, 5, 9, 13, 14, 10, 6, 2, 3, 7, 11, 15, 12, 8, 4]
POS = [0] * N_DEV
for _p, _d in enumerate(PERM):
    POS[_d] = _p


def kernel(x, w_mat, scale_x, scale_w):
    perm_in = jnp.array(PERM, dtype=jnp.int32)
    pos_in = jnp.array(POS, dtype=jnp.int32)

    def body(x_ref, w_ref, sx_ref, sw_ref, perm_ref, pos_ref, out_ref,
             send_f, recv_f, send_b, recv_b,
             sf_sems, rf_sems, sb_sems, rb_sems):
        d = lax.axis_index("i")
        p = pos_ref[d]
        nxt = perm_ref[(p + 1) % N_DEV]
        prv = perm_ref[(p + N_DEV - 1) % N_DEV]

        barrier_sem = pltpu.get_barrier_semaphore()
        for nbr in (prv, nxt):
            pl.semaphore_signal(
                barrier_sem, inc=1,
                device_id=(nbr,), device_id_type=pl.DeviceIdType.MESH,
            )
        pl.semaphore_wait(barrier_sem, 2)

        def dot_half(t, half):
            r = perm_ref[(p + t) % N_DEV] * CHUNK
            xc = x_ref[pl.ds(r, CHUNK), :]
            wc = w_ref[:, half * NH:(half + 1) * NH]
            return jnp.dot(xc, wc, preferred_element_type=jnp.float32)

        def mk(sbuf, rbuf, ssems, rsems, s, k, dev):
            return pltpu.make_async_remote_copy(
                src_ref=sbuf.at[s, k],
                dst_ref=rbuf.at[s, k],
                send_sem=ssems.at[s * B + k],
                recv_sem=rsems.at[s * B + k],
                device_id=(dev,),
                device_id_type=pl.DeviceIdType.MESH,
            )

        def mk_f(s, k):
            return mk(send_f, recv_f, sf_sems, rf_sems, s, k, nxt)

        def mk_b(s, k):
            return mk(send_b, recv_b, sb_sems, rb_sems, s, k, prv)

        pf = dot_half(N_DEV - 1, 0)
        pb = dot_half(1, 1)
        for k in range(B):
            sl = slice(k * SUB, (k + 1) * SUB)
            send_f[0, k] = pf[:, sl].astype(jnp.bfloat16)
            mk_f(0, k).start()
            send_b[0, k] = pb[:, sl].astype(jnp.bfloat16)
            mk_b(0, k).start()

        for s in range(1, N_HOP):
            pf = dot_half(N_DEV - 1 - s, 0)
            pb = dot_half(s + 1, 1)
            for k in range(B):
                sl = slice(k * SUB, (k + 1) * SUB)
                mk_f(s - 1, k).wait_recv()
                send_f[s, k] = (
                    recv_f[s - 1, k].astype(jnp.float32) + pf[:, sl]
                ).astype(jnp.bfloat16)
                mk_f(s, k).start()
                mk_b(s - 1, k).wait_recv()
                send_b[s, k] = (
                    recv_b[s - 1, k].astype(jnp.float32) + pb[:, sl]
                ).astype(jnp.bfloat16)
                mk_b(s, k).start()

        scale = sx_ref[0] * sw_ref[0]
        pf = dot_half(0, 0)
        pb = dot_half(0, 1)
        for k in range(B):
            sl = slice(k * SUB, (k + 1) * SUB)
            mk_f(N_HOP - 1, k).wait_recv()
            a = recv_f[N_HOP - 1, k].astype(jnp.float32) + pf[:, sl]
            y = a * scale
            out_ref[:, k * SUB:(k + 1) * SUB] = y * jax.nn.sigmoid(y)
            mk_b(N_HOP - 1, k).wait_recv()
            a = recv_b[N_HOP - 1, k].astype(jnp.float32) + pb[:, sl]
            y = a * scale
            out_ref[:, NH + k * SUB:NH + (k + 1) * SUB] = y * jax.nn.sigmoid(y)

        for s in range(N_HOP):
            for k in range(B):
                mk_f(s, k).wait_send()
                mk_b(s, k).wait_send()

    return pl.pallas_call(
        body,
        out_shape=jax.ShapeDtypeStruct((CHUNK, N), jnp.float32),
        in_specs=[
            pl.BlockSpec(memory_space=pltpu.VMEM),
            pl.BlockSpec(memory_space=pltpu.VMEM),
            pl.BlockSpec(memory_space=pltpu.SMEM),
            pl.BlockSpec(memory_space=pltpu.SMEM),
            pl.BlockSpec(memory_space=pltpu.SMEM),
            pl.BlockSpec(memory_space=pltpu.SMEM),
        ],
        out_specs=pl.BlockSpec(memory_space=pltpu.VMEM),
        scratch_shapes=[
            pltpu.VMEM((N_HOP, B, CHUNK, SUB), jnp.bfloat16),
            pltpu.VMEM((N_HOP, B, CHUNK, SUB), jnp.bfloat16),
            pltpu.VMEM((N_HOP, B, CHUNK, SUB), jnp.bfloat16),
            pltpu.VMEM((N_HOP, B, CHUNK, SUB), jnp.bfloat16),
            pltpu.SemaphoreType.DMA((N_HOP * B,)),
            pltpu.SemaphoreType.DMA((N_HOP * B,)),
            pltpu.SemaphoreType.DMA((N_HOP * B,)),
            pltpu.SemaphoreType.DMA((N_HOP * B,)),
        ],
        compiler_params=pltpu.CompilerParams(
            collective_id=0,
            vmem_limit_bytes=64 * 1024 * 1024,
        ),
    )(x, w_mat, scale_x, scale_w, perm_in, pos_in)


# device time: 97925 ns/iter; 2.1617x vs baseline; 1.0008x over previous
import jax
import jax.numpy as jnp
from jax import lax
from jax.experimental import pallas as pl
from jax.experimental.pallas import tpu as pltpu

N_DEV = 16
M = 4096
K_SHARD = 256
N = 2048
CHUNK = M // N_DEV
NH = N // 2
B = 4
SUB = NH // B
N_HOP = N_DEV - 1

PERM = [0, 1, 5, 9, 13, 14, 10, 6, 2, 3, 7, 11, 15, 12, 8, 4]
POS = [0] * N_DEV
for _p, _d in enumerate(PERM):
    POS[_d] = _p


def kernel(x, w_mat, scale_x, scale_w):
    perm_in = jnp.array(PERM, dtype=jnp.int32)
    pos_in = jnp.array(POS, dtype=jnp.int32)

    def body(x_ref, w_ref, sx_ref, sw_ref, perm_ref, pos_ref, out_ref,
             send_f, recv_f, send_b, recv_b,
             sf_sems, rf_sems, sb_sems, rb_sems):
        d = lax.axis_index("i")
        p = pos_ref[d]
        nxt = perm_ref[(p + 1) % N_DEV]
        prv = perm_ref[(p + N_DEV - 1) % N_DEV]

        barrier_sem = pltpu.get_barrier_semaphore()
        for nbr in (prv, nxt):
            pl.semaphore_signal(
                barrier_sem, inc=1,
                device_id=(nbr,), device_id_type=pl.DeviceIdType.MESH,
            )
        pl.semaphore_wait(barrier_sem, 2)

        def dot_half(t, half):
            r = perm_ref[(p + t) % N_DEV] * CHUNK
            xc = x_ref[pl.ds(r, CHUNK), :]
            wc = w_ref[:, half * NH:(half + 1) * NH]
            return jnp.dot(xc, wc, preferred_element_type=jnp.float32)

        def mk(sbuf, rbuf, ssems, rsems, s, k, dev):
            return pltpu.make_async_remote_copy(
                src_ref=sbuf.at[s, k],
                dst_ref=rbuf.at[s, k],
                send_sem=ssems.at[k],
                recv_sem=rsems.at[s * B + k],
                device_id=(dev,),
                device_id_type=pl.DeviceIdType.MESH,
            )

        def mk_f(s, k):
            return mk(send_f, recv_f, sf_sems, rf_sems, s, k, nxt)

        def mk_b(s, k):
            return mk(send_b, recv_b, sb_sems, rb_sems, s, k, prv)

        pf = dot_half(N_DEV - 1, 0)
        pb = dot_half(1, 1)
        for k in range(B):
            sl = slice(k * SUB, (k + 1) * SUB)
            send_f[0, k] = pf[:, sl].astype(jnp.bfloat16)
            mk_f(0, k).start()
            send_b[0, k] = pb[:, sl].astype(jnp.bfloat16)
            mk_b(0, k).start()

        for s in range(1, N_HOP):
            pf = dot_half(N_DEV - 1 - s, 0)
            pb = dot_half(s + 1, 1)
            for k in range(B):
                sl = slice(k * SUB, (k + 1) * SUB)
                mk_f(s - 1, k).wait_recv()
                send_f[s, k] = (
                    recv_f[s - 1, k].astype(jnp.float32) + pf[:, sl]
                ).astype(jnp.bfloat16)
                mk_f(s - 1, k).wait_send()
                mk_f(s, k).start()
                mk_b(s - 1, k).wait_recv()
                send_b[s, k] = (
                    recv_b[s - 1, k].astype(jnp.float32) + pb[:, sl]
                ).astype(jnp.bfloat16)
                mk_b(s - 1, k).wait_send()
                mk_b(s, k).start()

        scale = sx_ref[0] * sw_ref[0]
        pf = dot_half(0, 0)
        pb = dot_half(0, 1)
        for k in range(B):
            sl = slice(k * SUB, (k + 1) * SUB)
            mk_f(N_HOP - 1, k).wait_recv()
            a = recv_f[N_HOP - 1, k].astype(jnp.float32) + pf[:, sl]
            y = a * scale
            out_ref[:, k * SUB:(k + 1) * SUB] = y * jax.nn.sigmoid(y)
            mk_b(N_HOP - 1, k).wait_recv()
            a = recv_b[N_HOP - 1, k].astype(jnp.float32) + pb[:, sl]
            y = a * scale
            out_ref[:, NH + k * SUB:NH + (k + 1) * SUB] = y * jax.nn.sigmoid(y)

        for k in range(B):
            mk_f(N_HOP - 1, k).wait_send()
            mk_b(N_HOP - 1, k).wait_send()

    return pl.pallas_call(
        body,
        out_shape=jax.ShapeDtypeStruct((CHUNK, N), jnp.float32),
        in_specs=[
            pl.BlockSpec(memory_space=pltpu.VMEM),
            pl.BlockSpec(memory_space=pltpu.VMEM),
            pl.BlockSpec(memory_space=pltpu.SMEM),
            pl.BlockSpec(memory_space=pltpu.SMEM),
            pl.BlockSpec(memory_space=pltpu.SMEM),
            pl.BlockSpec(memory_space=pltpu.SMEM),
        ],
        out_specs=pl.BlockSpec(memory_space=pltpu.VMEM),
        scratch_shapes=[
            pltpu.VMEM((N_HOP, B, CHUNK, SUB), jnp.bfloat16),
            pltpu.VMEM((N_HOP, B, CHUNK, SUB), jnp.bfloat16),
            pltpu.VMEM((N_HOP, B, CHUNK, SUB), jnp.bfloat16),
            pltpu.VMEM((N_HOP, B, CHUNK, SUB), jnp.bfloat16),
            pltpu.SemaphoreType.DMA((B,)),
            pltpu.SemaphoreType.DMA((N_HOP * B,)),
            pltpu.SemaphoreType.DMA((B,)),
            pltpu.SemaphoreType.DMA((N_HOP * B,)),
        ],
        compiler_params=pltpu.CompilerParams(
            collective_id=0,
            vmem_limit_bytes=64 * 1024 * 1024,
        ),
    )(x, w_mat, scale_x, scale_w, perm_in, pos_in)
